# baseline (speedup 1.0000x reference)
import functools

import jax
import jax.numpy as jnp
from jax.experimental import pallas as pl
from jax.experimental.pallas import tpu as pltpu


def _gemm_kernel(x_ref, w_ref, o_ref):
    o_ref[...] = jnp.dot(
        x_ref[...].astype(jnp.bfloat16),
        w_ref[...],
        preferred_element_type=jnp.float32,
    )


@functools.partial(jax.jit, static_argnames=("tm",))
def _solve(xp, m_param, tm=256):
    B, M, N = xp.shape
    K = m_param.shape[0]
    rows = B * M
    x2d = xp.reshape(rows, N)
    w = m_param.T.astype(jnp.bfloat16)

    tm = min(tm, rows)
    grid_m = pl.cdiv(rows, tm)

    out = pl.pallas_call(
        _gemm_kernel,
        out_shape=jax.ShapeDtypeStruct((rows, K), jnp.float32),
        grid=(grid_m,),
        in_specs=[
            pl.BlockSpec((tm, N), lambda i: (i, 0)),
            pl.BlockSpec((N, K), lambda i: (0, 0)),
        ],
        out_specs=pl.BlockSpec((tm, K), lambda i: (i, 0)),
        compiler_params=pltpu.CompilerParams(
            dimension_semantics=("parallel",),
            vmem_limit_bytes=48 << 20,
        ),
    )(x2d, w)
    return out.reshape(B, M, K)


def kernel(xp, m_param):
    return _solve(xp, m_param)


# speedup vs baseline: 1.0981x; 1.0981x over previous
import functools

import jax
import jax.numpy as jnp
from jax import lax
from jax.experimental import pallas as pl
from jax.experimental.pallas import tpu as pltpu


def _gemm_kernel(x_ref, w_ref, o_ref):
    o_ref[...] = lax.dot_general(
        x_ref[...].astype(jnp.bfloat16),
        w_ref[...],
        dimension_numbers=(((1,), (1,)), ((), ())),
        preferred_element_type=jnp.float32,
    )


@functools.partial(jax.jit, static_argnames=("tm",))
def _solve(xp, m_param, tm=512):
    B, M, N = xp.shape
    K = m_param.shape[0]
    rows = B * M
    x2d = xp.reshape(rows, N)
    w = m_param.astype(jnp.bfloat16)

    tm = min(tm, rows)
    grid_m = pl.cdiv(rows, tm)

    out = pl.pallas_call(
        _gemm_kernel,
        out_shape=jax.ShapeDtypeStruct((rows, K), jnp.float32),
        grid=(grid_m,),
        in_specs=[
            pl.BlockSpec((tm, N), lambda i: (i, 0)),
            pl.BlockSpec((K, N), lambda i: (0, 0)),
        ],
        out_specs=pl.BlockSpec((tm, K), lambda i: (i, 0)),
        compiler_params=pltpu.CompilerParams(
            dimension_semantics=("parallel",),
            vmem_limit_bytes=48 << 20,
        ),
    )(x2d, w)
    return out.reshape(B, M, K)


def kernel(xp, m_param):
    return _solve(xp, m_param)
